# Initial kernel scaffold; baseline (speedup 1.0000x reference)
import jax
import jax.numpy as jnp
from jax.experimental import pallas as pl


def kernel(anchor, delta, score):
    raise NotImplementedError("write your pallas kernel here")


# speedup vs baseline: 1.0015x; 1.0015x over previous
import functools

import jax
import jax.numpy as jnp
from jax import lax
from jax.experimental import pallas as pl
from jax.experimental.pallas import tpu as pltpu

H = 800.0
W = 800.0
SAMPLING = 16.0
TOPN_NMS = 12000
N_NMS = 2000
THR_NMS = 0.7
N = 20000


def _decode_body(anchor_ref, delta_ref, score_ref, roi_ref, score_m_ref):
    a0 = anchor_ref[0, :]
    a1 = anchor_ref[1, :]
    a2 = anchor_ref[2, :]
    a3 = anchor_ref[3, :]
    d0 = delta_ref[0, :]
    d1 = delta_ref[1, :]
    d2 = delta_ref[2, :]
    d3 = delta_ref[3, :]

    anc_w = a2 - a0
    anc_h = a3 - a1
    anc_ctrx = a0 + anc_w / 2.0
    anc_ctry = a1 + anc_h / 2.0
    ctr_x = d0 * anc_w + anc_ctrx
    ctr_y = d1 * anc_h + anc_ctry
    w = jnp.exp(d2) * anc_w
    h = jnp.exp(d3) * anc_h

    x1 = jnp.maximum(ctr_x - w / 2.0, 0.0)
    y1 = jnp.maximum(ctr_y - h / 2.0, 0.0)
    x2 = jnp.maximum(ctr_x + w / 2.0, 0.0)
    y2 = jnp.maximum(ctr_y + h / 2.0, 0.0)
    y2 = jnp.where(y2 > H, H - 1.0, y2)
    x2 = jnp.where(x2 > W, W - 1.0, x2)

    valid = ((x2 - x1) > SAMPLING) & ((y2 - y1) > SAMPLING)
    roi_ref[0, :] = x1
    roi_ref[1, :] = y1
    roi_ref[2, :] = x2
    roi_ref[3, :] = y2
    score_m_ref[0, :] = jnp.where(valid, score_ref[0, :], -jnp.inf)


@jax.jit
def _decode(anchor_t, delta_t, score_2d):
    n = anchor_t.shape[1]
    return pl.pallas_call(
        _decode_body,
        out_shape=(
            jax.ShapeDtypeStruct((4, n), jnp.float32),
            jax.ShapeDtypeStruct((1, n), jnp.float32),
        ),
    )(anchor_t, delta_t, score_2d)


def _nms_keep_ref(boxes, thr):
    n = boxes.shape[0]
    areas = (boxes[:, 2] - boxes[:, 0]) * (boxes[:, 3] - boxes[:, 1])
    idxs = jnp.arange(n)

    def body(i, suppressed):
        xx1 = jnp.maximum(boxes[i, 0], boxes[:, 0])
        yy1 = jnp.maximum(boxes[i, 1], boxes[:, 1])
        xx2 = jnp.minimum(boxes[i, 2], boxes[:, 2])
        yy2 = jnp.minimum(boxes[i, 3], boxes[:, 3])
        inter = jnp.maximum(xx2 - xx1, 0.0) * jnp.maximum(yy2 - yy1, 0.0)
        iou = inter / (areas[i] + areas - inter + 1e-9)
        new_sup = (iou >= thr) & (idxs > i)
        return jnp.where(suppressed[i], suppressed, suppressed | new_sup)

    suppressed = jax.lax.fori_loop(0, n, body, jnp.zeros((n,), dtype=bool))
    return ~suppressed


def kernel(anchor, delta, score):
    roi_t, score_m = _decode(anchor.T, delta.T, score[None, :])
    score_m = score_m[0]
    order = jnp.argsort(-score_m)[:TOPN_NMS]
    boxes = roi_t.T[order]
    keep = _nms_keep_ref(boxes, THR_NMS)
    sel = jnp.nonzero(keep, size=N_NMS, fill_value=0)[0]
    return boxes[sel]


# speedup vs baseline: 187.7597x; 187.4871x over previous
import functools

import jax
import jax.numpy as jnp
from jax import lax
from jax.experimental import pallas as pl
from jax.experimental.pallas import tpu as pltpu

H = 800.0
W = 800.0
SAMPLING = 16.0
TOPN_NMS = 12000
N_NMS = 2000
THR_NMS = 0.7
N = 20000


def _decode_body(anchor_ref, delta_ref, score_ref, roi_ref, score_m_ref):
    a0 = anchor_ref[0, :]
    a1 = anchor_ref[1, :]
    a2 = anchor_ref[2, :]
    a3 = anchor_ref[3, :]
    d0 = delta_ref[0, :]
    d1 = delta_ref[1, :]
    d2 = delta_ref[2, :]
    d3 = delta_ref[3, :]

    anc_w = a2 - a0
    anc_h = a3 - a1
    anc_ctrx = a0 + anc_w / 2.0
    anc_ctry = a1 + anc_h / 2.0
    ctr_x = d0 * anc_w + anc_ctrx
    ctr_y = d1 * anc_h + anc_ctry
    w = jnp.exp(d2) * anc_w
    h = jnp.exp(d3) * anc_h

    x1 = jnp.maximum(ctr_x - w / 2.0, 0.0)
    y1 = jnp.maximum(ctr_y - h / 2.0, 0.0)
    x2 = jnp.maximum(ctr_x + w / 2.0, 0.0)
    y2 = jnp.maximum(ctr_y + h / 2.0, 0.0)
    y2 = jnp.where(y2 > H, H - 1.0, y2)
    x2 = jnp.where(x2 > W, W - 1.0, x2)

    valid = ((x2 - x1) > SAMPLING) & ((y2 - y1) > SAMPLING)
    roi_ref[0, :] = x1
    roi_ref[1, :] = y1
    roi_ref[2, :] = x2
    roi_ref[3, :] = y2
    score_m_ref[0, :] = jnp.where(valid, score_ref[0, :], -jnp.inf)


@jax.jit
def _decode(anchor_t, delta_t, score_2d):
    n = anchor_t.shape[1]
    return pl.pallas_call(
        _decode_body,
        out_shape=(
            jax.ShapeDtypeStruct((4, n), jnp.float32),
            jax.ShapeDtypeStruct((1, n), jnp.float32),
        ),
    )(anchor_t, delta_t, score_2d)


def _make_nms(npad, tile, interpret=False):
    nt = npad // tile

    def body(boxes_t_ref, boxes_c_ref, keep_ref, act_r, act_c, cond_ref):
        act_r[...] = boxes_t_ref[...]
        act_c[...] = boxes_c_ref[...]

        def tile_body(i, carry):
            sl = pl.ds(i * tile, tile)
            xi1 = act_r[0:1, sl]
            yi1 = act_r[1:2, sl]
            xi2 = act_r[2:3, sl]
            yi2 = act_r[3:4, sl]
            ai = (xi2 - xi1) * (yi2 - yi1)
            ti1 = act_c[sl, 0:1]
            ti2 = act_c[sl, 1:2]
            ti3 = act_c[sl, 2:3]
            ti4 = act_c[sl, 3:4]
            ta = (ti3 - ti1) * (ti4 - ti2)

            def cross(j, sup):
                sj = pl.ds(j * tile, tile)
                xj1 = act_c[sj, 0:1]
                yj1 = act_c[sj, 1:2]
                xj2 = act_c[sj, 2:3]
                yj2 = act_c[sj, 3:4]
                aj = (xj2 - xj1) * (yj2 - yj1)
                xx1 = jnp.maximum(xj1, xi1)
                yy1 = jnp.maximum(yj1, yi1)
                xx2 = jnp.minimum(xj2, xi2)
                yy2 = jnp.minimum(yj2, yi2)
                inter = (jnp.maximum(xx2 - xx1, 0.0)
                         * jnp.maximum(yy2 - yy1, 0.0))
                iou = inter / (aj + ai - inter + 1e-9)
                hit = jnp.where(iou >= THR_NMS, 1.0, 0.0)
                return jnp.maximum(sup, jnp.max(hit, axis=0)[None, :])

            sup0 = lax.fori_loop(0, i, cross,
                                 jnp.zeros((1, tile), jnp.float32))

            xx1 = jnp.maximum(ti1, xi1)
            yy1 = jnp.maximum(ti2, yi1)
            xx2 = jnp.minimum(ti3, xi2)
            yy2 = jnp.minimum(ti4, yi2)
            inter = jnp.maximum(xx2 - xx1, 0.0) * jnp.maximum(yy2 - yy1, 0.0)
            iou = inter / (ta + ai - inter + 1e-9)
            rlt = (lax.broadcasted_iota(jnp.int32, (tile, tile), 0)
                   < lax.broadcasted_iota(jnp.int32, (tile, tile), 1))
            cond_ref[...] = jnp.where((iou >= THR_NMS) & rlt, 1.0, 0.0)

            def witer(wcarry):
                s, _ = wcarry
                active_col = (1.0 - s)[0][:, None]
                m = jnp.max(cond_ref[...] * active_col, axis=0)[None, :]
                s_new = jnp.maximum(sup0, m)
                changed = jnp.max(jnp.abs(s_new - s)) > 0.0
                return (s_new, changed)

            s_final, _ = lax.while_loop(lambda c: c[1], witer,
                                        (sup0, jnp.bool_(True)))

            keep_ref[0:1, sl] = jnp.where(s_final > 0.5, 0, 1)
            fac_r = 1.0 - s_final
            fac_c = fac_r[0][:, None]
            act_r[0:1, sl] = xi1 * fac_r
            act_r[1:2, sl] = yi1 * fac_r
            act_r[2:3, sl] = xi2 * fac_r
            act_r[3:4, sl] = yi2 * fac_r
            act_c[sl, 0:1] = ti1 * fac_c
            act_c[sl, 1:2] = ti2 * fac_c
            act_c[sl, 2:3] = ti3 * fac_c
            act_c[sl, 3:4] = ti4 * fac_c
            return carry

        lax.fori_loop(0, nt, tile_body, 0)

    def call(boxes_t, boxes_c):
        return pl.pallas_call(
            body,
            out_shape=jax.ShapeDtypeStruct((1, npad), jnp.int32),
            scratch_shapes=[
                pltpu.VMEM((4, npad), jnp.float32),
                pltpu.VMEM((npad, 4), jnp.float32),
                pltpu.VMEM((tile, tile), jnp.float32),
            ],
            interpret=interpret,
        )(boxes_t, boxes_c)

    return call


def _nms_keep_ref(boxes, thr):
    n = boxes.shape[0]
    areas = (boxes[:, 2] - boxes[:, 0]) * (boxes[:, 3] - boxes[:, 1])
    idxs = jnp.arange(n)

    def body(i, suppressed):
        xx1 = jnp.maximum(boxes[i, 0], boxes[:, 0])
        yy1 = jnp.maximum(boxes[i, 1], boxes[:, 1])
        xx2 = jnp.minimum(boxes[i, 2], boxes[:, 2])
        yy2 = jnp.minimum(boxes[i, 3], boxes[:, 3])
        inter = jnp.maximum(xx2 - xx1, 0.0) * jnp.maximum(yy2 - yy1, 0.0)
        iou = inter / (areas[i] + areas - inter + 1e-9)
        new_sup = (iou >= thr) & (idxs > i)
        return jnp.where(suppressed[i], suppressed, suppressed | new_sup)

    suppressed = jax.lax.fori_loop(0, n, body, jnp.zeros((n,), dtype=bool))
    return ~suppressed


NPAD_NMS = 12288
TILE_NMS = 512
_nms_call = _make_nms(NPAD_NMS, TILE_NMS)


def kernel(anchor, delta, score):
    roi_t, score_m = _decode(anchor.T, delta.T, score[None, :])
    score_m = score_m[0]
    order = jnp.argsort(-score_m)[:TOPN_NMS]
    boxes = roi_t.T[order]
    boxes_pad = jnp.zeros((NPAD_NMS, 4), jnp.float32).at[:TOPN_NMS].set(boxes)
    keep = _nms_call(boxes_pad.T, boxes_pad)[0, :TOPN_NMS]
    sel = jnp.nonzero(keep > 0, size=N_NMS, fill_value=0)[0]
    return boxes[sel]
